# baseline (speedup 1.0000x reference)
import functools

import jax
import jax.numpy as jnp
from jax import lax
from jax.experimental import pallas as pl
from jax.experimental.pallas import tpu as pltpu
from jax.experimental.pallas import tpu_sc as plsc

_NC = 2
_NS = 16
_NW = _NC * _NS

_CHUNK = 96
_RING = 5


def _hop_sc(n_nodes, n_edges, feat):
  ept = n_edges // _NW
  nchunk = ept // _CHUNK
  ring = _RING
  assert nchunk % ring == 0
  nblk = nchunk // ring
  rows_per_io = n_nodes // _NS
  io_piece = 125
  n_piece = rows_per_io // io_piece

  mesh = plsc.VectorSubcoreMesh(core_axis_name="c", subcore_axis_name="s")

  @functools.partial(
      pl.kernel,
      out_type=jax.ShapeDtypeStruct((_NC, n_nodes, feat), jnp.float32),
      mesh=mesh,
      scratch_types=[
          pltpu.VMEM((nchunk, _CHUNK), jnp.int32),
          pltpu.VMEM((nchunk, _CHUNK), jnp.int32),
          pltpu.VMEM((ring, _CHUNK, feat), jnp.float32),
          pltpu.VMEM((io_piece, feat), jnp.float32),
          pltpu.VMEM_SHARED((n_nodes + _CHUNK, feat), jnp.float32),
          [pltpu.SemaphoreType.DMA] * ring,
          [pltpu.SemaphoreType.DMA] * ring,
      ],
      compiler_params=pltpu.CompilerParams(use_tc_tiling_on_sc=False),
  )
  def hop(g_hbm, row2d_hbm, col2d_hbm, out_hbm,
          rowall, colall, gbuf, stage, acc, gsems, ssems):
    c = lax.axis_index("c")
    s = lax.axis_index("s")
    wid = s * _NC + c
    rbase = s * rows_per_io
    cbase = wid * nchunk

    pltpu.sync_copy(row2d_hbm.at[pl.ds(cbase, nchunk)], rowall)
    pltpu.sync_copy(col2d_hbm.at[pl.ds(cbase, nchunk)], colall)

    zf = jnp.zeros((16,), jnp.float32)

    def zrow(i, carry):
      for j in range(feat // 16):
        stage[i, pl.ds(j * 16, 16)] = zf
      return carry

    lax.fori_loop(0, io_piece, zrow, 0)
    for t in range(n_piece):
      pltpu.sync_copy(stage, acc.at[pl.ds(rbase + t * io_piece, io_piece)])
    plsc.subcore_barrier()

    def gather(k, slot):
      return pltpu.async_copy(g_hbm.at[rowall.at[k]], gbuf.at[slot],
                              gsems[slot])

    def scatter(k, slot):
      return pltpu.async_copy(gbuf.at[slot], acc.at[colall.at[k]],
                              ssems[slot], add=True)

    def wait_gather(slot):
      pltpu.make_async_copy(g_hbm.at[rowall.at[0]], gbuf.at[slot],
                            gsems[slot]).wait()

    def wait_scatter(slot):
      pltpu.make_async_copy(gbuf.at[slot], acc.at[colall.at[0]],
                            ssems[slot]).wait()

    for r in range(ring):
      gather(r, r)

    def body(blk, carry):
      base = blk * ring
      for r in range(ring):
        wait_gather(r)
        scatter(base + r, r)
      nbase = base + ring
      for r in range(ring):
        wait_scatter(r)
        gather(nbase + r, r)
      return carry

    lax.fori_loop(0, nblk - 1, body, 0)

    base = (nblk - 1) * ring
    for r in range(ring):
      wait_gather(r)
      scatter(base + r, r)
    for r in range(ring):
      wait_scatter(r)

    plsc.subcore_barrier()

    for t in range(n_piece):
      pltpu.sync_copy(acc.at[pl.ds(rbase + t * io_piece, io_piece)], stage)
      pltpu.sync_copy(stage,
                      out_hbm.at[c, pl.ds(rbase + t * io_piece, io_piece)])

  return hop


def _hist_sc(n_nodes, n_edges):
  ept = n_edges // _NW
  nchunk = ept // _CHUNK
  rows_per_tile = 1000

  mesh = plsc.VectorSubcoreMesh(core_axis_name="c", subcore_axis_name="s")

  @functools.partial(
      pl.kernel,
      out_type=jax.ShapeDtypeStruct((_NC * n_nodes,), jnp.float32),
      mesh=mesh,
      scratch_types=[
          pltpu.VMEM((ept // _CHUNK, _CHUNK), jnp.int32),
          pltpu.VMEM((_CHUNK,), jnp.float32),
          pltpu.VMEM((rows_per_tile + 8,), jnp.float32),
          pltpu.VMEM_SHARED((n_nodes + _CHUNK,), jnp.float32),
          [pltpu.SemaphoreType.DMA] * 5,
      ],
      compiler_params=pltpu.CompilerParams(use_tc_tiling_on_sc=False),
  )
  def hist(col2d_hbm, out_hbm, colall, ones_v, stage, acc, sems):
    c = lax.axis_index("c")
    s = lax.axis_index("s")
    wid = s * _NC + c
    rbase = s * rows_per_tile
    cbase = wid * nchunk

    pltpu.sync_copy(col2d_hbm.at[pl.ds(cbase, nchunk)], colall)

    for i in range(_CHUNK // 16):
      ones_v[pl.ds(i * 16, 16)] = jnp.full((16,), 1.0, jnp.float32)

    zf = jnp.zeros((16,), jnp.float32)

    def zrow(i, carry):
      stage[pl.ds(i * 16, 16)] = zf
      return carry

    lax.fori_loop(0, (rows_per_tile + 8) // 16, zrow, 0)

    @pl.when(s < n_nodes // rows_per_tile)
    def _():
      pltpu.sync_copy(stage.at[pl.ds(0, rows_per_tile)],
                      acc.at[pl.ds(rbase, rows_per_tile)])

    plsc.subcore_barrier()

    def body(blk, carry):
      base = blk * 5
      for r in range(5):
        pltpu.async_copy(ones_v, acc.at[colall.at[base + r]], sems[r],
                         add=True)
      for r in range(5):
        pltpu.make_async_copy(ones_v, acc.at[colall.at[0]], sems[r]).wait()
      return carry

    lax.fori_loop(0, nchunk // 5, body, 0)
    plsc.subcore_barrier()

    @pl.when(s < n_nodes // rows_per_tile)
    def _():
      pltpu.sync_copy(acc.at[pl.ds(rbase, rows_per_tile)],
                      stage.at[pl.ds(0, rows_per_tile)])
      pltpu.sync_copy(stage.at[pl.ds(0, rows_per_tile)],
                      out_hbm.at[pl.ds(c * n_nodes + rbase, rows_per_tile)])

  return hist


def _tc_prep(x_ref, wt_ref, hist_ref, u_ref, dinv_ref):
  deg = hist_ref[:, 0:1] + hist_ref[:, 1:2] + 1.0
  dinv = lax.rsqrt(deg)
  dinv_ref[...] = dinv
  y = jnp.dot(x_ref[...], wt_ref[...], preferred_element_type=jnp.float32)
  u_ref[...] = y * dinv


def _tc_mid(p_ref, u_ref, dinv_ref, w_ref):
  dinv = dinv_ref[...]
  w_ref[...] = (p_ref[0] + p_ref[1] + u_ref[...]) * (dinv * dinv)


def _tc_final(q_ref, w_ref, dinv_ref, b_ref, out_ref):
  t = (q_ref[0] + q_ref[1] + w_ref[...]) * dinv_ref[...] + b_ref[...]
  m = jnp.max(t, axis=1, keepdims=True)
  e = jnp.exp(t - m)
  lse = jnp.log(jnp.sum(e, axis=1, keepdims=True))
  out_ref[...] = t - m - lse


def kernel(x, edge_index, W, b):
  n, f_in = x.shape
  f_out = W.shape[0]
  e = edge_index.shape[1]

  e_pad = -e % (_CHUNK * _RING * _NW)
  e_tot = e + e_pad
  row_p = jnp.concatenate(
      [edge_index[0], jnp.zeros((e_pad,), jnp.int32)])
  col_p = jnp.concatenate(
      [edge_index[1],
       n + (jnp.arange(e_pad, dtype=jnp.int32) % _CHUNK)])
  row2d = row_p.reshape(e_tot // _CHUNK, _CHUNK)
  col2d = col_p.reshape(e_tot // _CHUNK, _CHUNK)
  wt = W.T
  b2 = b.reshape(1, f_out)

  hist_k = _hist_sc(n, e_tot)
  hop_k = _hop_sc(n, e_tot, f_out)

  hist = hist_k(col2d)
  hist_t = hist.reshape(_NC, n).T

  u, dinv = pl.pallas_call(
      _tc_prep,
      out_shape=(
          jax.ShapeDtypeStruct((n, f_out), jnp.float32),
          jax.ShapeDtypeStruct((n, 1), jnp.float32),
      ),
  )(x, wt, hist_t)

  p = hop_k(u, row2d, col2d)

  w = pl.pallas_call(
      _tc_mid,
      out_shape=jax.ShapeDtypeStruct((n, f_out), jnp.float32),
  )(p, u, dinv)

  q = hop_k(w, row2d, col2d)

  out = pl.pallas_call(
      _tc_final,
      out_shape=jax.ShapeDtypeStruct((n, f_out), jnp.float32),
  )(q, w, dinv, b2)

  return out


# speedup vs baseline: 1.1992x; 1.1992x over previous
import functools

import jax
import jax.numpy as jnp
from jax import lax
from jax.experimental import pallas as pl
from jax.experimental.pallas import tpu as pltpu
from jax.experimental.pallas import tpu_sc as plsc

_NC = 2
_NS = 16
_NW = _NC * _NS

_CHUNK = 80
_RING = 5


def _mega_sc(n_nodes, n_edges, feat):
  ept = n_edges // _NS
  nchunk = ept // _CHUNK
  ring = _RING
  assert nchunk % ring == 0
  nblk = nchunk // ring
  rows_per_io = n_nodes // _NS
  piece = 125
  n_piece = rows_per_io // piece

  mesh = plsc.VectorSubcoreMesh(core_axis_name="c", subcore_axis_name="s")

  @functools.partial(
      pl.kernel,
      out_type=jax.ShapeDtypeStruct((_NC * n_nodes, feat), jnp.float32),
      mesh=mesh,
      scratch_types=[
          pltpu.VMEM((nchunk, _CHUNK), jnp.int32),
          pltpu.VMEM((nchunk, _CHUNK), jnp.int32),
          pltpu.VMEM((ring, _CHUNK, feat), jnp.float32),
          pltpu.VMEM((piece, feat), jnp.float32),
          pltpu.VMEM((piece, feat), jnp.float32),
          pltpu.VMEM((piece, feat), jnp.float32),
          pltpu.VMEM_SHARED((n_nodes, feat), jnp.float32),
          pltpu.VMEM_SHARED((n_nodes, feat), jnp.float32),
          [pltpu.SemaphoreType.DMA] * _RING,
          [pltpu.SemaphoreType.DMA] * _RING,
      ],
      compiler_params=pltpu.CompilerParams(use_tc_tiling_on_sc=False),
  )
  def mega(uf_hbm, d2m_hbm, row2d_hbm, col2d_hbm, q_hbm,
           rowall, colall, gbuf, pa, pb, pc, acc, tab, gsems, ssems):
    c = lax.axis_index("c")
    s = lax.axis_index("s")
    rbase = s * rows_per_io
    cbase = s * nchunk

    pltpu.sync_copy(row2d_hbm.at[pl.ds(cbase, nchunk)], rowall)
    pltpu.sync_copy(col2d_hbm.at[pl.ds(cbase, nchunk)], colall)

    zf = jnp.zeros((16,), jnp.float32)

    def zrow(i, carry):
      for j in range(feat // 16):
        pb[i, pl.ds(j * 16, 16)] = zf
      return carry

    lax.fori_loop(0, piece, zrow, 0)
    for t in range(n_piece):
      lo = rbase + t * piece
      pltpu.sync_copy(uf_hbm.at[pl.ds(c * n_nodes + lo, piece)], pa)
      pltpu.sync_copy(pa, tab.at[pl.ds(lo, piece)])
      pltpu.sync_copy(pb, acc.at[pl.ds(lo, piece)])
    plsc.subcore_barrier()

    def edge_sweep():
      def gather(k, slot):
        pltpu.async_copy(tab.at[rowall.at[k]], gbuf.at[slot], gsems[slot])

      def scatter(k, slot):
        pltpu.async_copy(gbuf.at[slot], acc.at[colall.at[k]], ssems[slot],
                         add=True)

      def wait_gather(slot):
        pltpu.make_async_copy(tab.at[rowall.at[0]], gbuf.at[slot],
                              gsems[slot]).wait()

      def wait_scatter(slot):
        pltpu.make_async_copy(gbuf.at[slot], acc.at[colall.at[0]],
                              ssems[slot]).wait()

      for r in range(ring):
        gather(r, r)

      def body(blk, carry):
        base = blk * ring
        for r in range(ring):
          wait_gather(r)
          scatter(base + r, r)
        nbase = base + ring
        for r in range(ring):
          wait_scatter(r)
          gather(nbase + r, r)
        return carry

      lax.fori_loop(0, nblk - 1, body, 0)

      base = (nblk - 1) * ring
      for r in range(ring):
        wait_gather(r)
        scatter(base + r, r)
      for r in range(ring):
        wait_scatter(r)

    edge_sweep()
    plsc.subcore_barrier()

    for t in range(n_piece):
      lo = rbase + t * piece
      pltpu.sync_copy(acc.at[pl.ds(lo, piece)], pa)
      pltpu.sync_copy(uf_hbm.at[pl.ds(c * n_nodes + lo, piece)], pb)
      pltpu.sync_copy(d2m_hbm.at[pl.ds(lo, piece)], pc)

      def mrow(i, carry):
        for j in range(feat // 16):
          sl = pl.ds(j * 16, 16)
          pa[i, sl] = (pa[i, sl] + pb[i, sl]) * pc[i, sl]
        return carry

      lax.fori_loop(0, piece, mrow, 0)
      pltpu.sync_copy(pa, tab.at[pl.ds(lo, piece)])
      pltpu.sync_copy(pa, acc.at[pl.ds(lo, piece)])
    plsc.subcore_barrier()

    edge_sweep()
    plsc.subcore_barrier()

    for t in range(n_piece):
      lo = rbase + t * piece
      pltpu.sync_copy(acc.at[pl.ds(lo, piece)], pa)
      pltpu.sync_copy(pa, q_hbm.at[pl.ds(c * n_nodes + lo, piece)])

  return mega


def _hist_sc(n_nodes, n_edges):
  ept = n_edges // _NW
  nchunk = ept // _CHUNK
  rows_per_tile = 1000

  mesh = plsc.VectorSubcoreMesh(core_axis_name="c", subcore_axis_name="s")

  @functools.partial(
      pl.kernel,
      out_type=jax.ShapeDtypeStruct((_NC * n_nodes,), jnp.float32),
      mesh=mesh,
      scratch_types=[
          pltpu.VMEM((ept // _CHUNK, _CHUNK), jnp.int32),
          pltpu.VMEM((_CHUNK,), jnp.float32),
          pltpu.VMEM((rows_per_tile + 8,), jnp.float32),
          pltpu.VMEM_SHARED((n_nodes,), jnp.float32),
          [pltpu.SemaphoreType.DMA] * 5,
      ],
      compiler_params=pltpu.CompilerParams(use_tc_tiling_on_sc=False),
  )
  def hist(col2d_hbm, out_hbm, colall, ones_v, stage, acc, sems):
    c = lax.axis_index("c")
    s = lax.axis_index("s")
    wid = s * _NC + c
    rbase = s * rows_per_tile
    cbase = wid * nchunk

    pltpu.sync_copy(col2d_hbm.at[pl.ds(cbase, nchunk)], colall)

    for i in range(_CHUNK // 16):
      ones_v[pl.ds(i * 16, 16)] = jnp.full((16,), 1.0, jnp.float32)

    zf = jnp.zeros((16,), jnp.float32)

    def zrow(i, carry):
      stage[pl.ds(i * 16, 16)] = zf
      return carry

    lax.fori_loop(0, (rows_per_tile + 8) // 16, zrow, 0)

    @pl.when(s < n_nodes // rows_per_tile)
    def _():
      pltpu.sync_copy(stage.at[pl.ds(0, rows_per_tile)],
                      acc.at[pl.ds(rbase, rows_per_tile)])

    plsc.subcore_barrier()

    def body(blk, carry):
      base = blk * 5
      for r in range(5):
        pltpu.async_copy(ones_v, acc.at[colall.at[base + r]], sems[r],
                         add=True)
      for r in range(5):
        pltpu.make_async_copy(ones_v, acc.at[colall.at[0]], sems[r]).wait()
      return carry

    lax.fori_loop(0, nchunk // 5, body, 0)
    plsc.subcore_barrier()

    @pl.when(s < n_nodes // rows_per_tile)
    def _():
      pltpu.sync_copy(acc.at[pl.ds(rbase, rows_per_tile)],
                      stage.at[pl.ds(0, rows_per_tile)])
      pltpu.sync_copy(stage.at[pl.ds(0, rows_per_tile)],
                      out_hbm.at[pl.ds(c * n_nodes + rbase, rows_per_tile)])

  return hist


def _tc_prep(n, fh):
  def prep(x_ref, wt_ref, hist_ref, uf_ref, d2m_ref, dinv_ref):
    deg = hist_ref[:, 0:1] + hist_ref[:, 1:2] + 1.0
    dinv = lax.rsqrt(deg)
    dinv_ref[...] = dinv
    y = jnp.dot(x_ref[...], wt_ref[...], preferred_element_type=jnp.float32)
    u = y * dinv
    uf_ref[pl.ds(0, n), :] = u[:, 0:fh]
    uf_ref[pl.ds(n, n), :] = u[:, fh:2 * fh]
    d2m_ref[...] = jnp.broadcast_to(dinv * dinv, (n, fh))
  return prep


def _tc_final(n, fh):
  def final(q_ref, dinv_ref, b_ref, out_ref):
    t = jnp.concatenate([q_ref[pl.ds(0, n), :], q_ref[pl.ds(n, n), :]],
                        axis=1)
    t = t * dinv_ref[...] + b_ref[...]
    m = jnp.max(t, axis=1, keepdims=True)
    e = jnp.exp(t - m)
    lse = jnp.log(jnp.sum(e, axis=1, keepdims=True))
    out_ref[...] = t - m - lse
  return final


def kernel(x, edge_index, W, b):
  n, f_in = x.shape
  f_out = W.shape[0]
  fh = f_out // _NC
  e = edge_index.shape[1]

  row2d = edge_index[0].reshape(e // _CHUNK, _CHUNK)
  col2d = edge_index[1].reshape(e // _CHUNK, _CHUNK)
  wt = W.T
  b2 = b.reshape(1, f_out)

  hist_k = _hist_sc(n, e)
  mega_k = _mega_sc(n, e, fh)

  hist = hist_k(col2d)
  hist_t = hist.reshape(_NC, n).T

  uf, d2m, dinv = pl.pallas_call(
      _tc_prep(n, fh),
      out_shape=(
          jax.ShapeDtypeStruct((_NC * n, fh), jnp.float32),
          jax.ShapeDtypeStruct((n, fh), jnp.float32),
          jax.ShapeDtypeStruct((n, 1), jnp.float32),
      ),
  )(x, wt, hist_t)

  q = mega_k(uf, d2m, row2d, col2d)

  out = pl.pallas_call(
      _tc_final(n, fh),
      out_shape=jax.ShapeDtypeStruct((n, f_out), jnp.float32),
  )(q, dinv, b2)

  return out


# speedup vs baseline: 1.4650x; 1.2217x over previous
import functools

import jax
import jax.numpy as jnp
from jax import lax
from jax.experimental import pallas as pl
from jax.experimental.pallas import tpu as pltpu
from jax.experimental.pallas import tpu_sc as plsc

_NC = 2
_NS = 16
_NW = _NC * _NS

_CHUNK = 80
_RING = 5


def _hop_sc(n_nodes, n_edges, feat):
  ept = n_edges // _NW
  nchunk = ept // _CHUNK
  ring = _RING
  assert nchunk % ring == 0
  nblk = nchunk // ring
  rows_per_io = n_nodes // _NS
  io_piece = 125
  n_piece = rows_per_io // io_piece

  mesh = plsc.VectorSubcoreMesh(core_axis_name="c", subcore_axis_name="s")

  @functools.partial(
      pl.kernel,
      out_type=jax.ShapeDtypeStruct((_NC, n_nodes, feat), jnp.float32),
      mesh=mesh,
      scratch_types=[
          pltpu.VMEM((nchunk, _CHUNK), jnp.int32),
          pltpu.VMEM((nchunk, _CHUNK), jnp.int32),
          pltpu.VMEM((ring, _CHUNK, feat), jnp.float32),
          pltpu.VMEM((io_piece, feat), jnp.float32),
          pltpu.VMEM_SHARED((n_nodes + _CHUNK, feat), jnp.float32),
          [pltpu.SemaphoreType.DMA] * ring,
          [pltpu.SemaphoreType.DMA] * ring,
      ],
      compiler_params=pltpu.CompilerParams(use_tc_tiling_on_sc=False),
  )
  def hop(g_hbm, row2d_hbm, col2d_hbm, out_hbm,
          rowall, colall, gbuf, stage, acc, gsems, ssems):
    c = lax.axis_index("c")
    s = lax.axis_index("s")
    wid = s * _NC + c
    rbase = s * rows_per_io
    cbase = wid * nchunk

    pltpu.sync_copy(row2d_hbm.at[pl.ds(cbase, nchunk)], rowall)
    pltpu.sync_copy(col2d_hbm.at[pl.ds(cbase, nchunk)], colall)

    zf = jnp.zeros((16,), jnp.float32)

    def zrow(i, carry):
      for j in range(feat // 16):
        stage[i, pl.ds(j * 16, 16)] = zf
      return carry

    lax.fori_loop(0, io_piece, zrow, 0)
    for t in range(n_piece):
      pltpu.sync_copy(stage, acc.at[pl.ds(rbase + t * io_piece, io_piece)])
    plsc.subcore_barrier()

    def gather(k, slot):
      return pltpu.async_copy(g_hbm.at[rowall.at[k]], gbuf.at[slot],
                              gsems[slot])

    def scatter(k, slot):
      return pltpu.async_copy(gbuf.at[slot], acc.at[colall.at[k]],
                              ssems[slot], add=True)

    def wait_gather(slot):
      pltpu.make_async_copy(g_hbm.at[rowall.at[0]], gbuf.at[slot],
                            gsems[slot]).wait()

    def wait_scatter(slot):
      pltpu.make_async_copy(gbuf.at[slot], acc.at[colall.at[0]],
                            ssems[slot]).wait()

    for r in range(ring):
      gather(r, r)

    def body(blk, carry):
      base = blk * ring
      for r in range(ring):
        wait_gather(r)
        scatter(base + r, r)
      nbase = base + ring
      for r in range(ring):
        wait_scatter(r)
        gather(nbase + r, r)
      return carry

    lax.fori_loop(0, nblk - 1, body, 0)

    base = (nblk - 1) * ring
    for r in range(ring):
      wait_gather(r)
      scatter(base + r, r)
    for r in range(ring):
      wait_scatter(r)

    plsc.subcore_barrier()

    for t in range(n_piece):
      pltpu.sync_copy(acc.at[pl.ds(rbase + t * io_piece, io_piece)], stage)
      pltpu.sync_copy(stage,
                      out_hbm.at[c, pl.ds(rbase + t * io_piece, io_piece)])

  return hop


def _hist_sc(n_nodes, n_edges):
  ept = n_edges // _NW
  nchunk = ept // _CHUNK
  rows_per_tile = 1000

  mesh = plsc.VectorSubcoreMesh(core_axis_name="c", subcore_axis_name="s")

  @functools.partial(
      pl.kernel,
      out_type=jax.ShapeDtypeStruct((_NC * n_nodes,), jnp.float32),
      mesh=mesh,
      scratch_types=[
          pltpu.VMEM((ept // _CHUNK, _CHUNK), jnp.int32),
          pltpu.VMEM((_CHUNK,), jnp.float32),
          pltpu.VMEM((rows_per_tile + 8,), jnp.float32),
          pltpu.VMEM_SHARED((n_nodes + _CHUNK,), jnp.float32),
          [pltpu.SemaphoreType.DMA] * 5,
      ],
      compiler_params=pltpu.CompilerParams(use_tc_tiling_on_sc=False),
  )
  def hist(col2d_hbm, out_hbm, colall, ones_v, stage, acc, sems):
    c = lax.axis_index("c")
    s = lax.axis_index("s")
    wid = s * _NC + c
    rbase = s * rows_per_tile
    cbase = wid * nchunk

    pltpu.sync_copy(col2d_hbm.at[pl.ds(cbase, nchunk)], colall)

    for i in range(_CHUNK // 16):
      ones_v[pl.ds(i * 16, 16)] = jnp.full((16,), 1.0, jnp.float32)

    zf = jnp.zeros((16,), jnp.float32)

    def zrow(i, carry):
      stage[pl.ds(i * 16, 16)] = zf
      return carry

    lax.fori_loop(0, (rows_per_tile + 8) // 16, zrow, 0)

    @pl.when(s < n_nodes // rows_per_tile)
    def _():
      pltpu.sync_copy(stage.at[pl.ds(0, rows_per_tile)],
                      acc.at[pl.ds(rbase, rows_per_tile)])

    plsc.subcore_barrier()

    def body(blk, carry):
      base = blk * 5
      for r in range(5):
        pltpu.async_copy(ones_v, acc.at[colall.at[base + r]], sems[r],
                         add=True)
      for r in range(5):
        pltpu.make_async_copy(ones_v, acc.at[colall.at[0]], sems[r]).wait()
      return carry

    lax.fori_loop(0, nchunk // 5, body, 0)
    plsc.subcore_barrier()

    @pl.when(s < n_nodes // rows_per_tile)
    def _():
      pltpu.sync_copy(acc.at[pl.ds(rbase, rows_per_tile)],
                      stage.at[pl.ds(0, rows_per_tile)])
      pltpu.sync_copy(stage.at[pl.ds(0, rows_per_tile)],
                      out_hbm.at[pl.ds(c * n_nodes + rbase, rows_per_tile)])

  return hist


def _tc_matmul(x_ref, wt_ref, y_ref):
  y_ref[...] = jnp.dot(x_ref[...], wt_ref[...],
                       preferred_element_type=jnp.float32)


def _tc_scale(y_ref, hist_ref, u_ref, dinv_ref):
  deg = hist_ref[:, 0:1] + hist_ref[:, 1:2] + 1.0
  dinv = lax.rsqrt(deg)
  dinv_ref[...] = dinv
  u_ref[...] = y_ref[...] * dinv


def _tc_mid(p_ref, u_ref, dinv_ref, w_ref):
  dinv = dinv_ref[...]
  w_ref[...] = (p_ref[0] + p_ref[1] + u_ref[...]) * (dinv * dinv)


def _tc_final(q_ref, w_ref, dinv_ref, b_ref, out_ref):
  t = (q_ref[0] + q_ref[1] + w_ref[...]) * dinv_ref[...] + b_ref[...]
  m = jnp.max(t, axis=1, keepdims=True)
  e = jnp.exp(t - m)
  lse = jnp.log(jnp.sum(e, axis=1, keepdims=True))
  out_ref[...] = t - m - lse


def kernel(x, edge_index, W, b):
  n, f_in = x.shape
  f_out = W.shape[0]
  e = edge_index.shape[1]

  e_pad = -e % (_CHUNK * _RING * _NW)
  e_tot = e + e_pad
  row_p = jnp.concatenate(
      [edge_index[0], jnp.zeros((e_pad,), jnp.int32)])
  col_p = jnp.concatenate(
      [edge_index[1],
       n + (jnp.arange(e_pad, dtype=jnp.int32) % _CHUNK)])
  row2d = row_p.reshape(e_tot // _CHUNK, _CHUNK)
  col2d = col_p.reshape(e_tot // _CHUNK, _CHUNK)
  wt = W.T
  b2 = b.reshape(1, f_out)

  hist_k = _hist_sc(n, e_tot)
  hop_k = _hop_sc(n, e_tot, f_out)

  hist = hist_k(col2d)
  y = pl.pallas_call(
      _tc_matmul,
      out_shape=jax.ShapeDtypeStruct((n, f_out), jnp.float32),
  )(x, wt)
  hist_t = hist.reshape(_NC, n).T

  u, dinv = pl.pallas_call(
      _tc_scale,
      out_shape=(
          jax.ShapeDtypeStruct((n, f_out), jnp.float32),
          jax.ShapeDtypeStruct((n, 1), jnp.float32),
      ),
  )(y, hist_t)

  p = hop_k(u, row2d, col2d)

  w = pl.pallas_call(
      _tc_mid,
      out_shape=jax.ShapeDtypeStruct((n, f_out), jnp.float32),
  )(p, u, dinv)

  q = hop_k(w, row2d, col2d)

  out = pl.pallas_call(
      _tc_final,
      out_shape=jax.ShapeDtypeStruct((n, f_out), jnp.float32),
  )(q, w, dinv, b2)

  return out


# speedup vs baseline: 1.4706x; 1.0038x over previous
import functools

import jax
import jax.numpy as jnp
from jax import lax
from jax.experimental import pallas as pl
from jax.experimental.pallas import tpu as pltpu
from jax.experimental.pallas import tpu_sc as plsc

_NC = 2
_NS = 16
_NW = _NC * _NS

_CHUNK = 80
_RING = 5


def _hop_sc(n_nodes, n_edges, feat):
  ept = n_edges // _NW
  nchunk = ept // _CHUNK
  ring = _RING
  assert nchunk % ring == 0
  nblk = nchunk // ring
  rows_per_io = n_nodes // _NS
  io_piece = 125
  n_piece = rows_per_io // io_piece

  mesh = plsc.VectorSubcoreMesh(core_axis_name="c", subcore_axis_name="s")

  @functools.partial(
      pl.kernel,
      out_type=jax.ShapeDtypeStruct((_NC, n_nodes, feat), jnp.float32),
      mesh=mesh,
      scratch_types=[
          pltpu.VMEM((nchunk, _CHUNK), jnp.int32),
          pltpu.VMEM((nchunk, _CHUNK), jnp.int32),
          pltpu.VMEM((ring, _CHUNK, feat), jnp.float32),
          pltpu.VMEM((io_piece, feat), jnp.float32),
          pltpu.VMEM_SHARED((n_nodes + _CHUNK, feat), jnp.float32),
          [pltpu.SemaphoreType.DMA] * ring,
          [pltpu.SemaphoreType.DMA] * ring,
      ],
      compiler_params=pltpu.CompilerParams(use_tc_tiling_on_sc=False),
  )
  def hop(g_hbm, row2d_hbm, col2d_hbm, out_hbm,
          rowall, colall, gbuf, stage, acc, gsems, ssems):
    c = lax.axis_index("c")
    s = lax.axis_index("s")
    wid = s * _NC + c
    rbase = s * rows_per_io
    cbase = wid * nchunk

    pltpu.sync_copy(row2d_hbm.at[pl.ds(cbase, nchunk)], rowall)
    pltpu.sync_copy(col2d_hbm.at[pl.ds(cbase, nchunk)], colall)

    zf = jnp.zeros((16,), jnp.float32)

    def zrow(i, carry):
      for j in range(feat // 16):
        stage[i, pl.ds(j * 16, 16)] = zf
      return carry

    lax.fori_loop(0, io_piece, zrow, 0)
    for t in range(n_piece):
      pltpu.sync_copy(stage, acc.at[pl.ds(rbase + t * io_piece, io_piece)])
    plsc.subcore_barrier()

    def gather(k, slot):
      return pltpu.async_copy(g_hbm.at[rowall.at[k]], gbuf.at[slot],
                              gsems[slot])

    def scatter(k, slot):
      return pltpu.async_copy(gbuf.at[slot], acc.at[colall.at[k]],
                              ssems[slot], add=True)

    def wait_gather(slot):
      pltpu.make_async_copy(g_hbm.at[rowall.at[0]], gbuf.at[slot],
                            gsems[slot]).wait()

    def wait_scatter(slot):
      pltpu.make_async_copy(gbuf.at[slot], acc.at[colall.at[0]],
                            ssems[slot]).wait()

    for r in range(ring):
      gather(r, r)

    def body(blk, carry):
      base = blk * ring
      for r in range(ring):
        wait_gather(r)
        scatter(base + r, r)
      nbase = base + ring
      for r in range(ring):
        wait_scatter(r)
        gather(nbase + r, r)
      return carry

    lax.fori_loop(0, nblk - 1, body, 0)

    base = (nblk - 1) * ring
    for r in range(ring):
      wait_gather(r)
      scatter(base + r, r)
    for r in range(ring):
      wait_scatter(r)

    plsc.subcore_barrier()

    for t in range(n_piece):
      pltpu.sync_copy(acc.at[pl.ds(rbase + t * io_piece, io_piece)], stage)
      pltpu.sync_copy(stage,
                      out_hbm.at[c, pl.ds(rbase + t * io_piece, io_piece)])

  return hop


def _hist_sc(n_nodes, n_edges):
  ept = n_edges // _NW
  nchunk = ept // _CHUNK
  rows_per_tile = 1000

  mesh = plsc.VectorSubcoreMesh(core_axis_name="c", subcore_axis_name="s")

  @functools.partial(
      pl.kernel,
      out_type=jax.ShapeDtypeStruct((_NC * n_nodes,), jnp.float32),
      mesh=mesh,
      scratch_types=[
          pltpu.VMEM((ept // _CHUNK, _CHUNK), jnp.int32),
          pltpu.VMEM((_CHUNK,), jnp.float32),
          pltpu.VMEM((rows_per_tile + 8,), jnp.float32),
          pltpu.VMEM_SHARED((n_nodes + _CHUNK,), jnp.float32),
          [pltpu.SemaphoreType.DMA] * 5,
      ],
      compiler_params=pltpu.CompilerParams(use_tc_tiling_on_sc=False),
  )
  def hist(col2d_hbm, out_hbm, colall, ones_v, stage, acc, sems):
    c = lax.axis_index("c")
    s = lax.axis_index("s")
    wid = s * _NC + c
    rbase = s * rows_per_tile
    cbase = wid * nchunk

    pltpu.sync_copy(col2d_hbm.at[pl.ds(cbase, nchunk)], colall)

    for i in range(_CHUNK // 16):
      ones_v[pl.ds(i * 16, 16)] = jnp.full((16,), 1.0, jnp.float32)

    zf = jnp.zeros((16,), jnp.float32)

    def zrow(i, carry):
      stage[pl.ds(i * 16, 16)] = zf
      return carry

    lax.fori_loop(0, (rows_per_tile + 8) // 16, zrow, 0)

    @pl.when(s < n_nodes // rows_per_tile)
    def _():
      pltpu.sync_copy(stage.at[pl.ds(0, rows_per_tile)],
                      acc.at[pl.ds(rbase, rows_per_tile)])

    plsc.subcore_barrier()

    def body(blk, carry):
      base = blk * 5
      for r in range(5):
        pltpu.async_copy(ones_v, acc.at[colall.at[base + r]], sems[r],
                         add=True)
      for r in range(5):
        pltpu.make_async_copy(ones_v, acc.at[colall.at[0]], sems[r]).wait()
      return carry

    lax.fori_loop(0, nchunk // 5, body, 0)
    plsc.subcore_barrier()

    @pl.when(s < n_nodes // rows_per_tile)
    def _():
      pltpu.sync_copy(acc.at[pl.ds(rbase, rows_per_tile)],
                      stage.at[pl.ds(0, rows_per_tile)])
      pltpu.sync_copy(stage.at[pl.ds(0, rows_per_tile)],
                      out_hbm.at[pl.ds(c * n_nodes + rbase, rows_per_tile)])

  return hist


def _tc_prep(x_ref, wt_ref, hist_ref, u_ref, dinv_ref):
  deg = hist_ref[:, 0:1] + hist_ref[:, 1:2] + 1.0
  dinv = lax.rsqrt(deg)
  dinv_ref[...] = dinv
  y = jnp.dot(x_ref[...], wt_ref[...], preferred_element_type=jnp.float32)
  u_ref[...] = y * dinv


def _tc_mid(p_ref, u_ref, dinv_ref, w_ref):
  dinv = dinv_ref[...]
  w_ref[...] = (p_ref[0] + p_ref[1] + u_ref[...]) * (dinv * dinv)


def _tc_final(q_ref, w_ref, dinv_ref, b_ref, out_ref):
  t = (q_ref[0] + q_ref[1] + w_ref[...]) * dinv_ref[...] + b_ref[...]
  m = jnp.max(t, axis=1, keepdims=True)
  e = jnp.exp(t - m)
  lse = jnp.log(jnp.sum(e, axis=1, keepdims=True))
  out_ref[...] = t - m - lse


def kernel(x, edge_index, W, b):
  n, f_in = x.shape
  f_out = W.shape[0]
  e = edge_index.shape[1]

  e_pad = -e % (_CHUNK * _RING * _NW)
  e_tot = e + e_pad
  row_p = jnp.concatenate(
      [edge_index[0], jnp.zeros((e_pad,), jnp.int32)])
  col_p = jnp.concatenate(
      [edge_index[1],
       n + (jnp.arange(e_pad, dtype=jnp.int32) % _CHUNK)])
  row2d = row_p.reshape(e_tot // _CHUNK, _CHUNK)
  col2d = col_p.reshape(e_tot // _CHUNK, _CHUNK)
  wt = W.T
  b2 = b.reshape(1, f_out)

  hist_k = _hist_sc(n, e_tot)
  hop_k = _hop_sc(n, e_tot, f_out)

  hist = hist_k(col2d)
  hist_t = hist.reshape(_NC, n).T

  u, dinv = pl.pallas_call(
      _tc_prep,
      out_shape=(
          jax.ShapeDtypeStruct((n, f_out), jnp.float32),
          jax.ShapeDtypeStruct((n, 1), jnp.float32),
      ),
  )(x, wt, hist_t)

  p = hop_k(u, row2d, col2d)

  w = pl.pallas_call(
      _tc_mid,
      out_shape=jax.ShapeDtypeStruct((n, f_out), jnp.float32),
  )(p, u, dinv)

  q = hop_k(w, row2d, col2d)

  out = pl.pallas_call(
      _tc_final,
      out_shape=jax.ShapeDtypeStruct((n, f_out), jnp.float32),
  )(q, w, dinv, b2)

  return out
